# baseline (speedup 1.0000x reference)
import functools

import jax
import jax.numpy as jnp
from jax import lax
from jax.experimental import pallas as pl
from jax.experimental.pallas import tpu as pltpu
from jax.experimental.pallas import tpu_sc as plsc

N = 10000
E = 320000
D_IN = 128
D_H = 64
D_OUT = 32
EPS = 0.01

NC, NS = 2, 16
NW = NC * NS
EPW = E // NW
CHUNK = 80
NCHUNK = EPW // CHUNK

SLC = 624
TAIL_OFF = SLC * NS
TAIL = N - TAIL_OFF

ROWS_PER_SUB = N // NS
STG = SLC // 3


def _sc_mesh():
    return plsc.VectorSubcoreMesh(
        core_axis_name="c", subcore_axis_name="s",
        num_cores=NC, num_subcores=NS)


def _deg_sc(ei4):
    @functools.partial(
        pl.kernel,
        mesh=_sc_mesh(),
        compiler_params=pltpu.CompilerParams(
            use_tc_tiling_on_sc=False, needs_layout_passes=False),
        out_type=jax.ShapeDtypeStruct((NC, N), jnp.float32),
        scratch_types=[
            pltpu.VMEM((NCHUNK, CHUNK), jnp.int32),
            pltpu.VMEM((N,), jnp.float32),
            pltpu.VMEM((NS, SLC), jnp.float32),
            pltpu.VMEM((NS, TAIL), jnp.float32),
            pltpu.VMEM_SHARED((NS, N), jnp.float32),
        ],
    )
    def k(ei_hbm, out_hbm, didx, deg_loc, buf, tbuf, deg_sh):
        c = lax.axis_index("c")
        s = lax.axis_index("s")
        wid = c * NS + s

        def zl(i, carry):
            deg_loc[pl.ds(i * 16, 16)] = jnp.zeros((16,), jnp.float32)
            return carry

        lax.fori_loop(0, N // 16, zl, 0)
        pltpu.sync_copy(ei_hbm.at[1, wid], didx)
        ones16 = jnp.ones((16,), jnp.float32)

        def body(j, carry):
            for q in range(CHUNK // 16):
                idxv = didx[j, pl.ds(q * 16, 16)]
                plsc.addupdate_scatter(deg_loc, [idxv], ones16)
            return carry

        lax.fori_loop(0, NCHUNK, body, 0)
        pltpu.sync_copy(deg_loc, deg_sh.at[s])
        plsc.subcore_barrier()
        pltpu.sync_copy(deg_sh.at[:, pl.ds(s * SLC, SLC)], buf)

        def addrow(i, carry):
            v = buf[0, pl.ds(i * 16, 16)]
            for t in range(1, NS):
                v = v + buf[t, pl.ds(i * 16, 16)]
            buf[0, pl.ds(i * 16, 16)] = v
            return carry

        lax.fori_loop(0, SLC // 16, addrow, 0)
        pltpu.sync_copy(buf.at[0], out_hbm.at[c, pl.ds(s * SLC, SLC)])

        @pl.when(s == NS - 1)
        def _():
            pltpu.sync_copy(deg_sh.at[:, pl.ds(TAIL_OFF, TAIL)], tbuf)
            v = tbuf[0, pl.ds(0, TAIL)]
            for t in range(1, NS):
                v = v + tbuf[t, pl.ds(0, TAIL)]
            tbuf[0, pl.ds(0, TAIL)] = v
            pltpu.sync_copy(tbuf.at[0], out_hbm.at[c, pl.ds(TAIL_OFF, TAIL)])

    return k(ei4)


NB = 5


def _agg_sc(y, ei4, D):
    stage_y = False
    scratch = [
        pltpu.VMEM((NCHUNK, CHUNK), jnp.int32),
        pltpu.VMEM((NCHUNK, CHUNK), jnp.int32),
        [[pltpu.VMEM((CHUNK, D), jnp.float32) for _ in range(NB)]
         for _ in range(2)],
        pltpu.VMEM((STG, D), jnp.float32),
        pltpu.VMEM_SHARED((N, D), jnp.float32),
        [[pltpu.SemaphoreType.DMA for _ in range(NB)] for _ in range(2)],
        [[pltpu.SemaphoreType.DMA for _ in range(NB)] for _ in range(2)],
    ]
    if stage_y:
        scratch.append(pltpu.VMEM_SHARED((N, D), jnp.float32))

    @functools.partial(
        pl.kernel,
        mesh=_sc_mesh(),
        compiler_params=pltpu.CompilerParams(use_tc_tiling_on_sc=False),
        out_type=jax.ShapeDtypeStruct((N, max(NC * D, 128)), jnp.float32),
        scratch_types=scratch,
    )
    def k(y_hbm, ei_hbm, out_hbm, sidx, didx, rows, stage, agg_sh, gsem,
          ssem, *maybe_ysh):
        c = lax.axis_index("c")
        s = lax.axis_index("s")
        wid = c * NS + s
        y_src = maybe_ysh[0] if stage_y else y_hbm

        def zb(i, carry):
            for j in range(D // 16):
                stage[i, pl.ds(j * 16, 16)] = jnp.zeros((16,), jnp.float32)
            return carry

        lax.fori_loop(0, STG, zb, 0)
        for t in range(SLC // STG):
            pltpu.sync_copy(stage,
                            agg_sh.at[pl.ds(s * SLC + t * STG, STG)])

        @pl.when(s == NS - 1)
        def _():
            pltpu.sync_copy(stage.at[pl.ds(0, TAIL)],
                            agg_sh.at[pl.ds(TAIL_OFF, TAIL)])

        if stage_y:
            pltpu.sync_copy(y_hbm.at[pl.ds(s * SLC, SLC)], stage)
            pltpu.sync_copy(stage, y_src.at[pl.ds(s * SLC, SLC)])

            @pl.when(s == NS - 1)
            def _():
                pltpu.sync_copy(y_hbm.at[pl.ds(TAIL_OFF, TAIL)],
                                stage.at[pl.ds(0, TAIL)])
                pltpu.sync_copy(stage.at[pl.ds(0, TAIL)],
                                y_src.at[pl.ds(TAIL_OFF, TAIL)])

        pltpu.sync_copy(ei_hbm.at[0, wid], sidx)
        pltpu.sync_copy(ei_hbm.at[1, wid], didx)
        plsc.subcore_barrier()

        def visit(o, cur, nxt, gs_c, gs_n, ss_c, ss_n, wait_prev):
            for b in range(NB):
                j = o * NB + b
                pltpu.make_async_copy(
                    y_src.at[sidx.at[j]], cur[b], gs_c[b]).wait()
                pltpu.async_copy(
                    cur[b], agg_sh.at[didx.at[j]], ss_c[b], add=True)
                jn = j + NB

                @pl.when(jn < NCHUNK)
                def _():
                    if wait_prev:
                        pltpu.make_async_copy(
                            nxt[b], agg_sh.at[didx.at[j]], ss_n[b]).wait()
                    pltpu.async_copy(y_src.at[sidx.at[jn]], nxt[b], gs_n[b])

        for b in range(NB):
            pltpu.async_copy(y_src.at[sidx.at[b]], rows[0][b], gsem[0][b])
        visit(0, rows[0], rows[1], gsem[0], gsem[1], ssem[0], ssem[1], False)

        def body(o2, carry):
            visit(2 * o2 + 1, rows[1], rows[0], gsem[1], gsem[0],
                  ssem[1], ssem[0], True)
            visit(2 * o2 + 2, rows[0], rows[1], gsem[0], gsem[1],
                  ssem[0], ssem[1], True)
            return carry

        lax.fori_loop(0, (NCHUNK // NB - 1) // 2, body, 0)
        for p in range(2):
            for b in range(NB):
                pltpu.make_async_copy(
                    rows[p][b], agg_sh.at[didx.at[0]], ssem[p][b]).wait()
        plsc.subcore_barrier()
        for t in range(SLC // STG):
            off = s * SLC + t * STG
            pltpu.sync_copy(agg_sh.at[pl.ds(off, STG)], stage)
            pltpu.sync_copy(stage,
                            out_hbm.at[pl.ds(off, STG), pl.ds(c * D, D)])

        @pl.when(s == NS - 1)
        def _():
            pltpu.sync_copy(agg_sh.at[pl.ds(TAIL_OFF, TAIL)],
                            stage.at[pl.ds(0, TAIL)])
            pltpu.sync_copy(stage.at[pl.ds(0, TAIL)],
                            out_hbm.at[pl.ds(TAIL_OFF, TAIL), pl.ds(c * D, D)])

    return k(y, ei4)


def _dinv_col(deg_ref):
    degcol = lax.dot_general(
        deg_ref[...], jnp.ones((NC, 1), jnp.float32),
        (((0,), (0,)), ((), ())), preferred_element_type=jnp.float32)
    return lax.rsqrt(degcol + 1.0)


def _p1_body(x_ref, w_ref, deg_ref, y_ref):
    xw = jnp.dot(x_ref[...], w_ref[...], preferred_element_type=jnp.float32)
    y_ref[...] = xw * _dinv_col(deg_ref)


def _p3_body(y1_ref, agg_ref, deg_ref, b1_ref, w3_ref, y2_ref):
    dinv = _dinv_col(deg_ref)
    agg = agg_ref[...]
    h1 = jnp.maximum(
        (agg[:, :D_H] + agg[:, D_H:] + y1_ref[...]) * dinv + b1_ref[...],
        0.0)
    y2_ref[...] = jnp.dot(
        h1, w3_ref[...],
        preferred_element_type=jnp.float32) * dinv


def _p5_body(y2_ref, agg_ref, deg_ref, b3_ref, out_ref):
    dinv = _dinv_col(deg_ref)
    agg = agg_ref[...]
    h = jnp.maximum(
        (agg[:, :D_OUT] + agg[:, D_OUT:2 * D_OUT] + y2_ref[...]) * dinv
        + b3_ref[...], 0.0)
    inv_n = 1.0 / N
    mean = jnp.sum(h, axis=0, keepdims=True) * inv_n
    ctr = h - mean
    var = jnp.sum(ctr * ctr, axis=0, keepdims=True) * inv_n
    var = jnp.maximum(var, EPS)
    out_ref[...] = ctr / jnp.sqrt(var)


def kernel(x, edge_index, W1, b1, W3, b3):
    ei4 = edge_index.astype(jnp.int32).reshape(2, NW, NCHUNK, CHUNK)

    deg = _deg_sc(ei4)

    y1 = pl.pallas_call(
        _p1_body,
        out_shape=jax.ShapeDtypeStruct((N, D_H), jnp.float32),
    )(x, W1, deg)

    agg1 = _agg_sc(y1, ei4, D_H)

    y2 = pl.pallas_call(
        _p3_body,
        out_shape=jax.ShapeDtypeStruct((N, D_OUT), jnp.float32),
    )(y1, agg1, deg, b1.reshape(1, D_H), W3)

    agg2 = _agg_sc(y2, ei4, D_OUT)

    out = pl.pallas_call(
        _p5_body,
        out_shape=jax.ShapeDtypeStruct((N, D_OUT), jnp.float32),
    )(y2, agg2, deg, b3.reshape(1, D_OUT))

    return out


# speedup vs baseline: 1.0120x; 1.0120x over previous
import functools

import jax
import jax.numpy as jnp
from jax import lax
from jax.experimental import pallas as pl
from jax.experimental.pallas import tpu as pltpu
from jax.experimental.pallas import tpu_sc as plsc

N = 10000
E = 320000
D_IN = 128
D_H = 64
D_OUT = 32
EPS = 0.01

NC, NS = 2, 16
NW = NC * NS
EPW = E // NW
CHUNK = 80
NCHUNK = EPW // CHUNK

SLC = 624
TAIL_OFF = SLC * NS
TAIL = N - TAIL_OFF

ROWS_PER_SUB = N // NS
STG = SLC // 3


def _sc_mesh():
    return plsc.VectorSubcoreMesh(
        core_axis_name="c", subcore_axis_name="s",
        num_cores=NC, num_subcores=NS)


def _deg_sc(ei4):
    @functools.partial(
        pl.kernel,
        mesh=_sc_mesh(),
        compiler_params=pltpu.CompilerParams(
            use_tc_tiling_on_sc=False, needs_layout_passes=False),
        out_type=jax.ShapeDtypeStruct((NC, N), jnp.float32),
        scratch_types=[
            pltpu.VMEM((NCHUNK, CHUNK), jnp.int32),
            pltpu.VMEM((N,), jnp.float32),
            pltpu.VMEM((NS, SLC), jnp.float32),
            pltpu.VMEM((NS, TAIL), jnp.float32),
            pltpu.VMEM_SHARED((NS, N), jnp.float32),
        ],
    )
    def k(ei_hbm, out_hbm, didx, deg_loc, buf, tbuf, deg_sh):
        c = lax.axis_index("c")
        s = lax.axis_index("s")
        wid = c * NS + s

        def zl(i, carry):
            deg_loc[pl.ds(i * 16, 16)] = jnp.zeros((16,), jnp.float32)
            return carry

        lax.fori_loop(0, N // 16, zl, 0)
        pltpu.sync_copy(ei_hbm.at[1, wid], didx)
        ones16 = jnp.ones((16,), jnp.float32)

        def body(j, carry):
            for q in range(CHUNK // 16):
                idxv = didx[j, pl.ds(q * 16, 16)]
                plsc.addupdate_scatter(deg_loc, [idxv], ones16)
            return carry

        lax.fori_loop(0, NCHUNK, body, 0)
        pltpu.sync_copy(deg_loc, deg_sh.at[s])
        plsc.subcore_barrier()
        pltpu.sync_copy(deg_sh.at[:, pl.ds(s * SLC, SLC)], buf)

        def addrow(i, carry):
            v = buf[0, pl.ds(i * 16, 16)]
            for t in range(1, NS):
                v = v + buf[t, pl.ds(i * 16, 16)]
            buf[0, pl.ds(i * 16, 16)] = v
            return carry

        lax.fori_loop(0, SLC // 16, addrow, 0)
        pltpu.sync_copy(buf.at[0], out_hbm.at[c, pl.ds(s * SLC, SLC)])

        @pl.when(s == NS - 1)
        def _():
            pltpu.sync_copy(deg_sh.at[:, pl.ds(TAIL_OFF, TAIL)], tbuf)
            v = tbuf[0, pl.ds(0, TAIL)]
            for t in range(1, NS):
                v = v + tbuf[t, pl.ds(0, TAIL)]
            tbuf[0, pl.ds(0, TAIL)] = v
            pltpu.sync_copy(tbuf.at[0], out_hbm.at[c, pl.ds(TAIL_OFF, TAIL)])

    return k(ei4)


NB = 5


def _agg_sc(y, ei4, D):
    scratch = [
        pltpu.VMEM((NCHUNK, CHUNK), jnp.int32),
        pltpu.VMEM((NCHUNK, CHUNK), jnp.int32),
        [pltpu.VMEM((CHUNK, D), jnp.float32) for _ in range(NB)],
        pltpu.VMEM((SLC, D), jnp.float32),
        pltpu.VMEM_SHARED((N, D), jnp.float32),
        [pltpu.SemaphoreType.DMA for _ in range(NB)],
    ]

    @functools.partial(
        pl.kernel,
        mesh=_sc_mesh(),
        compiler_params=pltpu.CompilerParams(use_tc_tiling_on_sc=False),
        out_type=jax.ShapeDtypeStruct((N, max(NC * D, 128)), jnp.float32),
        scratch_types=scratch,
    )
    def k(y_hbm, ei_hbm, out_hbm, sidx, didx, rows, stage, agg_sh, gsem):
        c = lax.axis_index("c")
        s = lax.axis_index("s")
        wid = c * NS + s
        y_src = y_hbm

        def zb(i, carry):
            for j in range(D // 16):
                stage[i, pl.ds(j * 16, 16)] = jnp.zeros((16,), jnp.float32)
            return carry

        lax.fori_loop(0, SLC, zb, 0)
        pltpu.sync_copy(stage, agg_sh.at[pl.ds(s * SLC, SLC)])

        @pl.when(s == NS - 1)
        def _():
            pltpu.sync_copy(stage.at[pl.ds(0, TAIL)],
                            agg_sh.at[pl.ds(TAIL_OFF, TAIL)])

        pltpu.sync_copy(ei_hbm.at[0, wid], sidx)
        pltpu.sync_copy(ei_hbm.at[1, wid], didx)
        plsc.subcore_barrier()

        for b in range(NB):
            pltpu.async_copy(y_src.at[sidx.at[b]], rows[b], gsem[b])

        def body(o, carry):
            for b in range(NB):
                j = o * NB + b
                pltpu.make_async_copy(
                    y_src.at[sidx.at[j]], rows[b], gsem[b]).wait()
                pltpu.sync_copy(rows[b], agg_sh.at[didx.at[j]], add=True)
                jn = j + NB

                @pl.when(jn < NCHUNK)
                def _():
                    pltpu.async_copy(y_src.at[sidx.at[jn]], rows[b], gsem[b])

            return carry

        lax.fori_loop(0, NCHUNK // NB, body, 0)
        plsc.subcore_barrier()
        pltpu.sync_copy(agg_sh.at[pl.ds(s * SLC, SLC)], stage)
        pltpu.sync_copy(stage,
                        out_hbm.at[pl.ds(s * SLC, SLC), pl.ds(c * D, D)])

        @pl.when(s == NS - 1)
        def _():
            pltpu.sync_copy(agg_sh.at[pl.ds(TAIL_OFF, TAIL)],
                            stage.at[pl.ds(0, TAIL)])
            pltpu.sync_copy(stage.at[pl.ds(0, TAIL)],
                            out_hbm.at[pl.ds(TAIL_OFF, TAIL), pl.ds(c * D, D)])

    return k(y, ei4)


def _dinv_col(deg_ref):
    degcol = lax.dot_general(
        deg_ref[...], jnp.ones((NC, 1), jnp.float32),
        (((0,), (0,)), ((), ())), preferred_element_type=jnp.float32)
    return lax.rsqrt(degcol + 1.0)


def _p1_body(x_ref, w_ref, deg_ref, y_ref):
    xw = jnp.dot(x_ref[...], w_ref[...], preferred_element_type=jnp.float32)
    y_ref[...] = xw * _dinv_col(deg_ref)


def _p3_body(y1_ref, agg_ref, deg_ref, b1_ref, w3_ref, y2_ref):
    dinv = _dinv_col(deg_ref)
    agg = agg_ref[...]
    h1 = jnp.maximum(
        (agg[:, :D_H] + agg[:, D_H:] + y1_ref[...]) * dinv + b1_ref[...],
        0.0)
    y2_ref[...] = jnp.dot(
        h1, w3_ref[...],
        preferred_element_type=jnp.float32) * dinv


def _p5_body(y2_ref, agg_ref, deg_ref, b3_ref, out_ref):
    dinv = _dinv_col(deg_ref)
    agg = agg_ref[...]
    h = jnp.maximum(
        (agg[:, :D_OUT] + agg[:, D_OUT:2 * D_OUT] + y2_ref[...]) * dinv
        + b3_ref[...], 0.0)
    inv_n = 1.0 / N
    mean = jnp.sum(h, axis=0, keepdims=True) * inv_n
    ctr = h - mean
    var = jnp.sum(ctr * ctr, axis=0, keepdims=True) * inv_n
    var = jnp.maximum(var, EPS)
    out_ref[...] = ctr / jnp.sqrt(var)


def kernel(x, edge_index, W1, b1, W3, b3):
    ei4 = edge_index.astype(jnp.int32).reshape(2, NW, NCHUNK, CHUNK)

    deg = _deg_sc(ei4)

    y1 = pl.pallas_call(
        _p1_body,
        out_shape=jax.ShapeDtypeStruct((N, D_H), jnp.float32),
    )(x, W1, deg)

    agg1 = _agg_sc(y1, ei4, D_H)

    y2 = pl.pallas_call(
        _p3_body,
        out_shape=jax.ShapeDtypeStruct((N, D_OUT), jnp.float32),
    )(y1, agg1, deg, b1.reshape(1, D_H), W3)

    agg2 = _agg_sc(y2, ei4, D_OUT)

    out = pl.pallas_call(
        _p5_body,
        out_shape=jax.ShapeDtypeStruct((N, D_OUT), jnp.float32),
    )(y2, agg2, deg, b3.reshape(1, D_OUT))

    return out


# speedup vs baseline: 1.0183x; 1.0062x over previous
import functools

import jax
import jax.numpy as jnp
from jax import lax
from jax.experimental import pallas as pl
from jax.experimental.pallas import tpu as pltpu
from jax.experimental.pallas import tpu_sc as plsc

N = 10000
E = 320000
D_IN = 128
D_H = 64
D_OUT = 32
EPS = 0.01

NC, NS = 2, 16
NW = NC * NS
EPW = E // NW
CHUNK = 125
NCHUNK = EPW // CHUNK

SLC = 624
TAIL_OFF = SLC * NS
TAIL = N - TAIL_OFF

ROWS_PER_SUB = N // NS
STG = SLC // 2


def _sc_mesh():
    return plsc.VectorSubcoreMesh(
        core_axis_name="c", subcore_axis_name="s",
        num_cores=NC, num_subcores=NS)


def _deg_sc(ei4):
    @functools.partial(
        pl.kernel,
        mesh=_sc_mesh(),
        compiler_params=pltpu.CompilerParams(
            use_tc_tiling_on_sc=False, needs_layout_passes=False),
        out_type=jax.ShapeDtypeStruct((NC, N), jnp.float32),
        scratch_types=[
            pltpu.VMEM((NCHUNK, CHUNK), jnp.int32),
            pltpu.VMEM((N,), jnp.float32),
            pltpu.VMEM((NS, SLC), jnp.float32),
            pltpu.VMEM((NS, TAIL), jnp.float32),
            pltpu.VMEM_SHARED((NS, N), jnp.float32),
        ],
    )
    def k(ei_hbm, out_hbm, didx, deg_loc, buf, tbuf, deg_sh):
        c = lax.axis_index("c")
        s = lax.axis_index("s")
        wid = c * NS + s

        def zl(i, carry):
            deg_loc[pl.ds(i * 16, 16)] = jnp.zeros((16,), jnp.float32)
            return carry

        lax.fori_loop(0, N // 16, zl, 0)
        pltpu.sync_copy(ei_hbm.at[1, wid], didx)
        ones16 = jnp.ones((16,), jnp.float32)

        rem = CHUNK - (CHUNK // 16) * 16
        lane = lax.broadcasted_iota(jnp.int32, (16,), 0)

        def body(j, carry):
            for q in range(CHUNK // 16):
                idxv = didx[j, pl.ds(q * 16, 16)]
                plsc.addupdate_scatter(deg_loc, [idxv], ones16)
            if rem:
                idxv = didx[j, pl.ds(CHUNK - 16, 16)]
                plsc.addupdate_scatter(deg_loc, [idxv], ones16,
                                       mask=lane >= (16 - rem))
            return carry

        lax.fori_loop(0, NCHUNK, body, 0)
        pltpu.sync_copy(deg_loc, deg_sh.at[s])
        plsc.subcore_barrier()
        pltpu.sync_copy(deg_sh.at[:, pl.ds(s * SLC, SLC)], buf)

        def addrow(i, carry):
            v = buf[0, pl.ds(i * 16, 16)]
            for t in range(1, NS):
                v = v + buf[t, pl.ds(i * 16, 16)]
            buf[0, pl.ds(i * 16, 16)] = v
            return carry

        lax.fori_loop(0, SLC // 16, addrow, 0)
        pltpu.sync_copy(buf.at[0], out_hbm.at[c, pl.ds(s * SLC, SLC)])

        @pl.when(s == NS - 1)
        def _():
            pltpu.sync_copy(deg_sh.at[:, pl.ds(TAIL_OFF, TAIL)], tbuf)
            v = tbuf[0, pl.ds(0, TAIL)]
            for t in range(1, NS):
                v = v + tbuf[t, pl.ds(0, TAIL)]
            tbuf[0, pl.ds(0, TAIL)] = v
            pltpu.sync_copy(tbuf.at[0], out_hbm.at[c, pl.ds(TAIL_OFF, TAIL)])

    return k(ei4)


NB = 5


def _agg_sc(y, ei4, D):
    scratch = [
        pltpu.VMEM((NCHUNK, CHUNK), jnp.int32),
        pltpu.VMEM((NCHUNK, CHUNK), jnp.int32),
        [pltpu.VMEM((CHUNK, D), jnp.float32) for _ in range(NB)],
        pltpu.VMEM((STG, D), jnp.float32),
        pltpu.VMEM_SHARED((N, D), jnp.float32),
        [pltpu.SemaphoreType.DMA for _ in range(NB)],
    ]

    @functools.partial(
        pl.kernel,
        mesh=_sc_mesh(),
        compiler_params=pltpu.CompilerParams(use_tc_tiling_on_sc=False),
        out_type=jax.ShapeDtypeStruct((N, max(NC * D, 128)), jnp.float32),
        scratch_types=scratch,
    )
    def k(y_hbm, ei_hbm, out_hbm, sidx, didx, rows, stage, agg_sh, gsem):
        c = lax.axis_index("c")
        s = lax.axis_index("s")
        wid = c * NS + s
        y_src = y_hbm

        def zb(i, carry):
            for j in range(D // 16):
                stage[i, pl.ds(j * 16, 16)] = jnp.zeros((16,), jnp.float32)
            return carry

        lax.fori_loop(0, STG, zb, 0)
        for t in range(SLC // STG):
            pltpu.sync_copy(stage, agg_sh.at[pl.ds(s * SLC + t * STG, STG)])

        @pl.when(s == NS - 1)
        def _():
            pltpu.sync_copy(stage.at[pl.ds(0, TAIL)],
                            agg_sh.at[pl.ds(TAIL_OFF, TAIL)])

        pltpu.sync_copy(ei_hbm.at[0, wid], sidx)
        pltpu.sync_copy(ei_hbm.at[1, wid], didx)
        plsc.subcore_barrier()

        for b in range(NB):
            pltpu.async_copy(y_src.at[sidx.at[b]], rows[b], gsem[b])

        def body(o, carry):
            for b in range(NB):
                j = o * NB + b
                pltpu.make_async_copy(
                    y_src.at[sidx.at[j]], rows[b], gsem[b]).wait()
                pltpu.sync_copy(rows[b], agg_sh.at[didx.at[j]], add=True)
                jn = j + NB

                @pl.when(jn < NCHUNK)
                def _():
                    pltpu.async_copy(y_src.at[sidx.at[jn]], rows[b], gsem[b])

            return carry

        lax.fori_loop(0, NCHUNK // NB, body, 0)
        plsc.subcore_barrier()
        for t in range(SLC // STG):
            off = s * SLC + t * STG
            pltpu.sync_copy(agg_sh.at[pl.ds(off, STG)], stage)
            pltpu.sync_copy(stage,
                            out_hbm.at[pl.ds(off, STG), pl.ds(c * D, D)])

        @pl.when(s == NS - 1)
        def _():
            pltpu.sync_copy(agg_sh.at[pl.ds(TAIL_OFF, TAIL)],
                            stage.at[pl.ds(0, TAIL)])
            pltpu.sync_copy(stage.at[pl.ds(0, TAIL)],
                            out_hbm.at[pl.ds(TAIL_OFF, TAIL), pl.ds(c * D, D)])

    return k(y, ei4)


def _dinv_col(deg_ref):
    degcol = lax.dot_general(
        deg_ref[...], jnp.ones((NC, 1), jnp.float32),
        (((0,), (0,)), ((), ())), preferred_element_type=jnp.float32)
    return lax.rsqrt(degcol + 1.0)


def _p1_body(x_ref, w_ref, deg_ref, y_ref):
    xw = jnp.dot(x_ref[...], w_ref[...], preferred_element_type=jnp.float32)
    y_ref[...] = xw * _dinv_col(deg_ref)


def _p3_body(y1_ref, agg_ref, deg_ref, b1_ref, w3_ref, y2_ref):
    dinv = _dinv_col(deg_ref)
    agg = agg_ref[...]
    h1 = jnp.maximum(
        (agg[:, :D_H] + agg[:, D_H:] + y1_ref[...]) * dinv + b1_ref[...],
        0.0)
    y2_ref[...] = jnp.dot(
        h1, w3_ref[...],
        preferred_element_type=jnp.float32) * dinv


def _p5_body(y2_ref, agg_ref, deg_ref, b3_ref, out_ref):
    dinv = _dinv_col(deg_ref)
    agg = agg_ref[...]
    h = jnp.maximum(
        (agg[:, :D_OUT] + agg[:, D_OUT:2 * D_OUT] + y2_ref[...]) * dinv
        + b3_ref[...], 0.0)
    inv_n = 1.0 / N
    mean = jnp.sum(h, axis=0, keepdims=True) * inv_n
    ctr = h - mean
    var = jnp.sum(ctr * ctr, axis=0, keepdims=True) * inv_n
    var = jnp.maximum(var, EPS)
    out_ref[...] = ctr / jnp.sqrt(var)


def kernel(x, edge_index, W1, b1, W3, b3):
    ei4 = edge_index.astype(jnp.int32).reshape(2, NW, NCHUNK, CHUNK)

    deg = _deg_sc(ei4)

    y1 = pl.pallas_call(
        _p1_body,
        out_shape=jax.ShapeDtypeStruct((N, D_H), jnp.float32),
    )(x, W1, deg)

    agg1 = _agg_sc(y1, ei4, D_H)

    y2 = pl.pallas_call(
        _p3_body,
        out_shape=jax.ShapeDtypeStruct((N, D_OUT), jnp.float32),
    )(y1, agg1, deg, b1.reshape(1, D_H), W3)

    agg2 = _agg_sc(y2, ei4, D_OUT)

    out = pl.pallas_call(
        _p5_body,
        out_shape=jax.ShapeDtypeStruct((N, D_OUT), jnp.float32),
    )(y2, agg2, deg, b3.reshape(1, D_OUT))

    return out
